# baseline (speedup 1.0000x reference)
import functools
import math

import jax
import jax.numpy as jnp
from jax import lax
from jax.experimental import pallas as pl
from jax.experimental.pallas import tpu as pltpu
from jax.experimental.pallas import tpu_sc as plsc

N_LEVELS = 16
F = 2
LOG2_T = 19
T = 1 << LOG2_T
MASK = T - 1
BASE_RES = 16
SCALE = 1.5
P = 4 * 65536
OUT_D = 32

NC = 2
NS = 16
NW = NC * NS
PER_W = P // NW
B = 1024
SB = PER_W // B
NSTEP = B // 16
NG = 16 * B

RES = [float(math.floor(BASE_RES * (SCALE ** l))) for l in range(N_LEVELS)]
PY = -1640531535
PZ = 805459861

_mesh = plsc.VectorSubcoreMesh(core_axis_name="c", subcore_axis_name="s")


@functools.partial(
    pl.kernel,
    mesh=_mesh,
    out_type=jax.ShapeDtypeStruct((2 * N_LEVELS, P), jnp.float32),
    scratch_types=[
        pltpu.VMEM((B,), jnp.float32),
        pltpu.VMEM((B,), jnp.float32),
        pltpu.VMEM((B,), jnp.float32),
        pltpu.VMEM((NG,), jnp.int32),
        pltpu.VMEM((NG,), jnp.int32),
        pltpu.VMEM((8 * B,), jnp.float32),
        pltpu.VMEM((8 * B,), jnp.float32),
        pltpu.VMEM((NG,), jnp.float32),
        pltpu.VMEM((NG,), jnp.float32),
        pltpu.VMEM((2 * N_LEVELS, B), jnp.float32),
        pltpu.SemaphoreType.DMA,
        pltpu.SemaphoreType.DMA,
    ],
)
def _hash_encode(x0h, x1h, x2h, tabh, outh,
                 xs, ys, zs, idx0, idx1, w0, w1, rows0, rows1, encb,
                 semA, semB):
    wid = lax.axis_index("s") * NC + lax.axis_index("c")

    idx_bufs = (idx0, idx1)
    w_bufs = (w0, w1)
    row_bufs = (rows0, rows1)
    sems = (semA, semB)

    def build(l, j, _):
        off = j * 16
        res = RES[l]
        loff2 = 2 * l * T
        xv = xs[pl.ds(off, 16)] * res
        yv = ys[pl.ds(off, 16)] * res
        zv = zs[pl.ds(off, 16)] * res
        xi = xv.astype(jnp.int32)
        yi = yv.astype(jnp.int32)
        zi = zv.astype(jnp.int32)
        fx = xv - xi.astype(jnp.float32)
        fy = yv - yi.astype(jnp.float32)
        fz = zv - zi.astype(jnp.float32)
        hx = (xi, xi + 1)
        hy0 = yi * PY
        hy = (hy0, hy0 + PY)
        hz0 = zi * PZ
        hz = (hz0, hz0 + PZ)
        gx = (1.0 - fx, fx)
        gy = (1.0 - fy, fy)
        gz = (1.0 - fz, fz)
        idx_ref = idx_bufs[l % 2]
        w_ref = w_bufs[l % 2]
        for by in range(2):
            for bx in range(2):
                hxy = hx[bx] ^ hy[by]
                wxy = gx[bx] * gy[by]
                for bz in range(2):
                    c = bx + 2 * by + 4 * bz
                    h2 = ((hxy ^ hz[bz]) & MASK) * 2 + loff2
                    w = wxy * gz[bz]
                    idx_ref[pl.ds(2 * c * B + off, 16)] = h2
                    idx_ref[pl.ds((2 * c + 1) * B + off, 16)] = h2 + 1
                    w_ref[pl.ds(c * B + off, 16)] = w
        return 0

    def fire(l):
        return pltpu.async_copy(tabh.at[idx_bufs[l % 2]], row_bufs[l % 2],
                                sems[l % 2])

    def accum(l, j, _):
        off = j * 16
        w_ref = w_bufs[l % 2]
        rows_ref = row_bufs[l % 2]
        f0 = jnp.zeros((16,), jnp.float32)
        f1 = jnp.zeros((16,), jnp.float32)
        for c in range(8):
            wv = w_ref[pl.ds(c * B + off, 16)]
            r0 = rows_ref[pl.ds(2 * c * B + off, 16)]
            r1 = rows_ref[pl.ds((2 * c + 1) * B + off, 16)]
            f0 = f0 + wv * r0
            f1 = f1 + wv * r1
        encb[2 * l, pl.ds(off, 16)] = f0
        encb[2 * l + 1, pl.ds(off, 16)] = f1
        return 0

    def sub_block(sb, _):
        base = wid * PER_W + sb * B
        pltpu.sync_copy(x0h.at[pl.ds(base, B)], xs)
        pltpu.sync_copy(x1h.at[pl.ds(base, B)], ys)
        pltpu.sync_copy(x2h.at[pl.ds(base, B)], zs)

        def norm(j, _):
            off = j * 16
            xs[pl.ds(off, 16)] = xs[pl.ds(off, 16)] * 0.5 + 0.5
            ys[pl.ds(off, 16)] = ys[pl.ds(off, 16)] * 0.5 + 0.5
            zs[pl.ds(off, 16)] = zs[pl.ds(off, 16)] * 0.5 + 0.5
            return 0

        lax.fori_loop(0, NSTEP, norm, 0)

        lax.fori_loop(0, NSTEP, functools.partial(build, 0), 0)
        handles = {0: fire(0)}
        for l in range(1, N_LEVELS):
            lax.fori_loop(0, NSTEP, functools.partial(build, l), 0)
            handles[l] = fire(l)
            handles[l - 1].wait()
            lax.fori_loop(0, NSTEP, functools.partial(accum, l - 1), 0)
        handles[N_LEVELS - 1].wait()
        lax.fori_loop(0, NSTEP, functools.partial(accum, N_LEVELS - 1), 0)

        pltpu.sync_copy(encb, outh.at[:, pl.ds(base, B)])
        return 0

    lax.fori_loop(0, SB, sub_block, 0)


def _mm_body(enc_ref, w_ref, b_ref, o_ref):
    o_ref[...] = lax.dot_general(
        enc_ref[...], w_ref[...],
        dimension_numbers=(((0,), (1,)), ((), ())),
        preferred_element_type=jnp.float32,
    ) + b_ref[...]


_MB = 8192


@jax.jit
def _combine(enc_t, W, b):
    return pl.pallas_call(
        _mm_body,
        grid=(P // _MB,),
        in_specs=[
            pl.BlockSpec((2 * N_LEVELS, _MB), lambda i: (0, i)),
            pl.BlockSpec((OUT_D, 2 * N_LEVELS), lambda i: (0, 0)),
            pl.BlockSpec((1, OUT_D), lambda i: (0, 0)),
        ],
        out_specs=pl.BlockSpec((_MB, OUT_D), lambda i: (i, 0)),
        out_shape=jax.ShapeDtypeStruct((P, OUT_D), jnp.float32),
    )(enc_t, W, b.reshape(1, OUT_D))


@jax.jit
def kernel(x, table, W, b):
    Bx, N, D = x.shape
    xf = x.reshape(-1, D)
    xt = xf.T
    tab2 = table.reshape(N_LEVELS * T * F)
    enc_t = _hash_encode(xt[0], xt[1], xt[2], tab2)
    out = _combine(enc_t, W, b)
    return out.reshape(Bx, N, OUT_D)
